# baseline (speedup 1.0000x reference)
import jax
import jax.numpy as jnp
from jax.experimental import pallas as pl


def _add_pos_kernel(x_ref, pos_ref, o_ref):
    o_ref[...] = x_ref[...] + pos_ref[...]


def kernel(x, pos_table):
    B, L, D = x.shape
    x2 = x.reshape(B, L * D)
    pos_row = jax.lax.slice(pos_table, (0, 0), (L, D)).reshape(1, L * D)
    BB = 64
    out2 = pl.pallas_call(
        _add_pos_kernel,
        grid=(B // BB,),
        in_specs=[
            pl.BlockSpec((BB, L * D), lambda i: (i, 0)),
            pl.BlockSpec((1, L * D), lambda i: (0, 0)),
        ],
        out_specs=pl.BlockSpec((BB, L * D), lambda i: (i, 0)),
        out_shape=jax.ShapeDtypeStruct((B, L * D), x.dtype),
    )(x2, pos_row)
    return out2.reshape(B, L, D)


# speedup vs baseline: 1.0099x; 1.0099x over previous
import functools

import jax
import jax.numpy as jnp
from jax.experimental import pallas as pl
from jax.experimental.pallas import tpu as pltpu

_NBUF = 8
_CH = 32


def _body(x_hbm, pos_vmem, o_hbm, ibuf, obuf, in_sems, out_sems):
    n_rows = x_hbm.shape[0]
    n_chunks = n_rows // _CH

    def in_copy(c, slot):
        return pltpu.make_async_copy(
            x_hbm.at[pl.ds(c * _CH, _CH), :], ibuf.at[slot], in_sems.at[slot]
        )

    def out_copy(c, slot):
        return pltpu.make_async_copy(
            obuf.at[slot], o_hbm.at[pl.ds(c * _CH, _CH), :], out_sems.at[slot]
        )

    for k in range(_NBUF):
        in_copy(k, k).start()

    pos = pos_vmem[...]

    def step(c, carry):
        slot = jax.lax.rem(c, _NBUF)
        in_copy(c, slot).wait()

        @pl.when(c >= _NBUF)
        def _():
            out_copy(c - _NBUF, slot).wait()

        obuf[slot] = ibuf[slot] + pos
        out_copy(c, slot).start()

        @pl.when(c + _NBUF < n_chunks)
        def _():
            in_copy(c + _NBUF, slot).start()

        return carry

    jax.lax.fori_loop(0, n_chunks, step, 0)

    for k in range(_NBUF):
        c = n_chunks - _NBUF + k
        out_copy(c, jax.lax.rem(c, _NBUF)).wait()


def kernel(x, pos_table):
    B, L, D = x.shape
    LD = L * D
    x2 = x.reshape(B, LD)
    pos_row = jax.lax.slice(pos_table, (0, 0), (L, D)).reshape(1, LD)
    out2 = pl.pallas_call(
        _body,
        in_specs=[
            pl.BlockSpec(memory_space=pltpu.HBM),
            pl.BlockSpec(memory_space=pltpu.VMEM),
        ],
        out_specs=pl.BlockSpec(memory_space=pltpu.HBM),
        out_shape=jax.ShapeDtypeStruct((B, LD), x.dtype),
        scratch_shapes=[
            pltpu.VMEM((_NBUF, _CH, LD), jnp.float32),
            pltpu.VMEM((_NBUF, _CH, LD), jnp.float32),
            pltpu.SemaphoreType.DMA((_NBUF,)),
            pltpu.SemaphoreType.DMA((_NBUF,)),
        ],
    )(x2, pos_row)
    return out2.reshape(B, L, D)


# speedup vs baseline: 3.5374x; 3.5027x over previous
import jax
import jax.numpy as jnp
from jax.experimental import pallas as pl


def _add_pos_kernel(x_ref, pos_ref, o_ref):
    o_ref[...] = x_ref[...] + pos_ref[...]


def kernel(x, pos_table):
    B, L, D = x.shape
    LD = L * D
    xt = jnp.transpose(x, (1, 2, 0)).reshape(LD, B)
    pos_col = jax.lax.slice(pos_table, (0, 0), (L, D)).reshape(LD, 1)
    R = 512
    out_t = pl.pallas_call(
        _add_pos_kernel,
        grid=(LD // R,),
        in_specs=[
            pl.BlockSpec((R, B), lambda i: (i, 0)),
            pl.BlockSpec((R, 1), lambda i: (i, 0)),
        ],
        out_specs=pl.BlockSpec((R, B), lambda i: (i, 0)),
        out_shape=jax.ShapeDtypeStruct((LD, B), x.dtype),
    )(xt, pos_col)
    return jnp.transpose(out_t.reshape(L, D, B), (2, 0, 1))


# speedup vs baseline: 3.5392x; 1.0005x over previous
import jax
import jax.numpy as jnp
from jax.experimental import pallas as pl


def _add_pos_kernel(x_ref, pos_ref, o_ref):
    o_ref[...] = x_ref[...] + pos_ref[...]


def kernel(x, pos_table):
    B, L, D = x.shape
    LD = L * D
    xt = jnp.transpose(x, (1, 2, 0)).reshape(LD, B)
    pos_col = jax.lax.slice(pos_table, (0, 0), (L, D)).reshape(LD, 1)
    R = 800
    out_t = pl.pallas_call(
        _add_pos_kernel,
        grid=(LD // R,),
        in_specs=[
            pl.BlockSpec((R, B), lambda i: (i, 0)),
            pl.BlockSpec((R, 1), lambda i: (i, 0)),
        ],
        out_specs=pl.BlockSpec((R, B), lambda i: (i, 0)),
        out_shape=jax.ShapeDtypeStruct((LD, B), x.dtype),
    )(xt, pos_col)
    return jnp.transpose(out_t.reshape(L, D, B), (2, 0, 1))


# speedup vs baseline: 3.8330x; 1.0830x over previous
import jax
import jax.numpy as jnp
from jax.experimental import pallas as pl


def _add_pos_kernel(x_ref, pos_ref, o_ref):
    o_ref[...] = x_ref[...] + pos_ref[...][:, :, None]


def kernel(x, pos_table):
    B, L, D = x.shape
    xt = jnp.transpose(x, (1, 2, 0))
    pos = jax.lax.slice(pos_table, (0, 0), (L, D))
    LB = 8
    out_t = pl.pallas_call(
        _add_pos_kernel,
        grid=(L // LB,),
        in_specs=[
            pl.BlockSpec((LB, D, B), lambda i: (i, 0, 0)),
            pl.BlockSpec((LB, D), lambda i: (i, 0)),
        ],
        out_specs=pl.BlockSpec((LB, D, B), lambda i: (i, 0, 0)),
        out_shape=jax.ShapeDtypeStruct((L, D, B), x.dtype),
    )(xt, pos)
    return jnp.transpose(out_t, (2, 0, 1))
